# baseline (speedup 1.0000x reference)
import functools

import jax
import jax.numpy as jnp
from jax import lax
from jax.experimental import pallas as pl
from jax.experimental.pallas import tpu as pltpu
from jax.experimental.pallas import tpu_sc as plsc

D = 64
L = 16
NC = 2
NS = 16
NW = NC * NS

BLK = 128
NB = 3
BG = BLK // L


@jax.jit
def _sc_gather(idx2d, table):
    n_blocks, _ = idx2d.shape
    n_h = n_blocks * BLK // 16384
    n_bt = 16384 // BLK
    per_w = n_blocks // NW
    mesh = plsc.VectorSubcoreMesh(core_axis_name="c", subcore_axis_name="s")

    @functools.partial(
        pl.kernel,
        out_type=jax.ShapeDtypeStruct((n_h, D // 8, n_bt, 8, BLK),
                                      jnp.float32),
        mesh=mesh,
        compiler_params=pltpu.CompilerParams(
            needs_layout_passes=False, use_tc_tiling_on_sc=False),
        scratch_types=[
            pltpu.VMEM((NB, BLK), jnp.int32),
            pltpu.VMEM((NB, BLK, D), jnp.float32),
            pltpu.VMEM((D, BLK), jnp.float32),
            pltpu.VMEM((NB, D // 8, 1, 8, BLK), jnp.float32),
            pltpu.SemaphoreType.DMA((NB,)),
            pltpu.SemaphoreType.DMA((NB,)),
            pltpu.SemaphoreType.DMA((NB,)),
        ],
    )
    def k(idx_hbm, tab_hbm, out_hbm, idx_v, rows_v, s_v, t_v,
          isem, gsem, osem):
        wid = lax.axis_index("s") * NC + lax.axis_index("c")
        blk0 = wid * per_w

        def out_hslice(c):
            blk = blk0 + c
            h = blk // n_bt
            bt = lax.rem(blk, n_bt)
            return out_hbm.at[h, pl.ds(0, D // 8), pl.ds(bt, 1),
                              pl.ds(0, 8), pl.ds(0, BLK)]

        def start_idx(c, b):
            pltpu.async_copy(idx_hbm.at[blk0 + c], idx_v.at[b], isem.at[b])

        def wait_idx(c, b):
            pltpu.make_async_copy(idx_hbm.at[blk0 + c], idx_v.at[b],
                                  isem.at[b]).wait()

        def start_gather(b):
            pltpu.async_copy(tab_hbm.at[idx_v.at[b]], rows_v.at[b],
                             gsem.at[b])

        def wait_gather(b):
            pltpu.make_async_copy(tab_hbm.at[idx_v.at[b]], rows_v.at[b],
                                  gsem.at[b]).wait()

        def start_out(c, b):
            pltpu.async_copy(t_v.at[b], out_hslice(c), osem.at[b])

        def wait_out(c, b):
            pltpu.make_async_copy(t_v.at[b], out_hslice(c),
                                  osem.at[b]).wait()

        for b in range(NB):
            start_idx(b, b)
        wait_idx(0, 0)
        start_gather(0)

        lane = lax.broadcasted_iota(jnp.int32, (L,), 0)

        def blk_body(g, carry):
            b = lax.rem(g, NB)

            @pl.when(g + 1 < per_w)
            def _next_gather():
                b1 = lax.rem(g + 1, NB)
                wait_idx(g + 1, b1)
                start_gather(b1)

            wait_gather(b)

            @pl.when(g >= NB)
            def _reuse():
                wait_out(g - NB, b)

            scales = []
            rows16 = []
            for bg in range(BG):
                idx16 = idx_v[b, pl.ds(bg * L, L)]
                scales.append(jnp.where(idx16 == 0, 0.0, 1.0))
                rows16.append(bg * L + lane)
            rows_b = rows_v.at[b]

            def pass1(d, dvec):
                for bg in range(BG):
                    v = plsc.load_gather(rows_b, [rows16[bg], dvec])
                    s_v[d, pl.ds(bg * L, L)] = v * scales[bg]
                return (dvec + 1) & (D - 1)

            lax.fori_loop(0, D, pass1, lane, unroll=4)

            def pass2(c, cvec):
                for bg in range(BG):
                    v = plsc.load_gather(s_v, [cvec, rows16[bg]])
                    t_v[b, c // 8, 0, lax.rem(c, 8), pl.ds(bg * L, L)] = v
                return (cvec + 1) & (D - 1)

            lax.fori_loop(0, D, pass2, (D - lane) & (D - 1), unroll=4)

            start_out(g, b)

            @pl.when(g + NB < per_w)
            def _next_idx():
                start_idx(g + NB, b)

            return carry

        lax.fori_loop(0, per_w, blk_body, 0)

        for c in range(per_w - NB, per_w):
            wait_out(c, c % NB)

    return k(idx2d, table)


def kernel(event, table):
    nb, nh = event.shape
    idx2d = event.T.reshape(nb * nh // BLK, BLK)
    out5 = _sc_gather(idx2d, table)
    return out5.transpose(2, 4, 0, 1, 3).reshape(nb, nh, D)


# speedup vs baseline: 2.6079x; 2.6079x over previous
import functools

import jax
import jax.numpy as jnp
from jax import lax
from jax.experimental import pallas as pl
from jax.experimental.pallas import tpu as pltpu
from jax.experimental.pallas import tpu_sc as plsc

D = 64
L = 16
NC = 2
NS = 16
NW = NC * NS

BLK = 128
NB = 3
BG = BLK // L


@jax.jit
def _sc_gather(idx2d, table):
    n_blocks, _ = idx2d.shape
    n_h = n_blocks * BLK // 16384
    n_bt = 16384 // BLK
    per_w = n_blocks // NW
    mesh = plsc.VectorSubcoreMesh(core_axis_name="c", subcore_axis_name="s")

    @functools.partial(
        pl.kernel,
        out_type=jax.ShapeDtypeStruct((n_h, D // 8, n_bt, 8, BLK),
                                      jnp.float32),
        mesh=mesh,
        compiler_params=pltpu.CompilerParams(
            needs_layout_passes=False, use_tc_tiling_on_sc=False),
        scratch_types=[
            pltpu.VMEM((NB, BLK), jnp.int32),
            pltpu.VMEM((NB, BLK, D), jnp.float32),
            pltpu.VMEM((D, BLK), jnp.float32),
            pltpu.VMEM((NB, D // 8, 1, 8, BLK), jnp.float32),
            pltpu.SemaphoreType.DMA((NB,)),
            pltpu.SemaphoreType.DMA((NB,)),
            pltpu.SemaphoreType.DMA((NB,)),
        ],
    )
    def k(idx_hbm, tab_hbm, out_hbm, idx_v, rows_v, s_v, t_v,
          isem, gsem, osem):
        wid = lax.axis_index("s") * NC + lax.axis_index("c")
        blk0 = wid * per_w

        def out_hslice(c):
            blk = blk0 + c
            h = blk // n_bt
            bt = lax.rem(blk, n_bt)
            return out_hbm.at[h, pl.ds(0, D // 8), pl.ds(bt, 1),
                              pl.ds(0, 8), pl.ds(0, BLK)]

        def start_idx(c, b):
            pltpu.async_copy(idx_hbm.at[blk0 + c], idx_v.at[b], isem.at[b])

        def wait_idx(c, b):
            pltpu.make_async_copy(idx_hbm.at[blk0 + c], idx_v.at[b],
                                  isem.at[b]).wait()

        def start_gather(b):
            pltpu.async_copy(tab_hbm.at[idx_v.at[b]], rows_v.at[b],
                             gsem.at[b])

        def wait_gather(b):
            pltpu.make_async_copy(tab_hbm.at[idx_v.at[b]], rows_v.at[b],
                                  gsem.at[b]).wait()

        def start_out(c, b):
            pltpu.async_copy(t_v.at[b], out_hslice(c), osem.at[b])

        def wait_out(c, b):
            pltpu.make_async_copy(t_v.at[b], out_hslice(c),
                                  osem.at[b]).wait()

        for b in range(NB):
            start_idx(b, b)
        wait_idx(0, 0)
        start_gather(0)

        lane = lax.broadcasted_iota(jnp.int32, (L,), 0)

        def blk_body(g, carry):
            b = lax.rem(g, NB)

            @pl.when(g + 1 < per_w)
            def _next_gather():
                b1 = lax.rem(g + 1, NB)
                wait_idx(g + 1, b1)
                start_gather(b1)

            wait_gather(b)

            @pl.when(g >= NB)
            def _reuse():
                wait_out(g - NB, b)

            scales = []
            rows16 = []
            for bg in range(BG):
                idx16 = idx_v[b, pl.ds(bg * L, L)]
                scales.append(jnp.where(idx16 == 0, 0.0, 1.0))
                rows16.append(bg * L + lane)
            rows_b = rows_v.at[b]

            def pass1(d, dvec):
                vs = [plsc.load_gather(rows_b, [rows16[bg], dvec])
                      for bg in range(BG)]
                for bg in range(BG):
                    s_v[d, pl.ds(bg * L, L)] = vs[bg] * scales[bg]
                return (dvec + 1) & (D - 1)

            lax.fori_loop(0, D, pass1, lane, unroll=4)

            def pass2(c, cvec):
                vs = [plsc.load_gather(s_v, [cvec, rows16[bg]])
                      for bg in range(BG)]
                for bg in range(BG):
                    t_v[b, c // 8, 0, lax.rem(c, 8), pl.ds(bg * L, L)] = (
                        vs[bg])
                return (cvec + 1) & (D - 1)

            lax.fori_loop(0, D, pass2, (D - lane) & (D - 1), unroll=4)

            start_out(g, b)

            @pl.when(g + NB < per_w)
            def _next_idx():
                start_idx(g + NB, b)

            return carry

        lax.fori_loop(0, per_w, blk_body, 0)

        for c in range(per_w - NB, per_w):
            wait_out(c, c % NB)

    return k(idx2d, table)


def kernel(event, table):
    nb, nh = event.shape
    idx2d = event.T.reshape(nb * nh // BLK, BLK)
    out5 = _sc_gather(idx2d, table)
    return out5.transpose(2, 4, 0, 1, 3).reshape(nb, nh, D)


# speedup vs baseline: 2.6273x; 1.0074x over previous
import functools

import jax
import jax.numpy as jnp
from jax import lax
from jax.experimental import pallas as pl
from jax.experimental.pallas import tpu as pltpu
from jax.experimental.pallas import tpu_sc as plsc

D = 64
L = 16
NC = 2
NS = 16
NW = NC * NS

BLK = 128
NB = 3
BG = BLK // L


@jax.jit
def _sc_gather(idx2d, table):
    n_blocks, _ = idx2d.shape
    n_h = n_blocks * BLK // 16384
    n_bt = 16384 // BLK
    per_w = n_blocks // NW
    mesh = plsc.VectorSubcoreMesh(core_axis_name="c", subcore_axis_name="s")

    @functools.partial(
        pl.kernel,
        out_type=jax.ShapeDtypeStruct((n_h, D // 8, n_bt, 8, BLK),
                                      jnp.float32),
        mesh=mesh,
        compiler_params=pltpu.CompilerParams(
            needs_layout_passes=False, use_tc_tiling_on_sc=False),
        scratch_types=[
            pltpu.VMEM((NB, BLK), jnp.int32),
            pltpu.VMEM((NB, BLK, D), jnp.float32),
            pltpu.VMEM((D, BLK), jnp.float32),
            pltpu.VMEM((NB, D // 8, 1, 8, BLK), jnp.float32),
            pltpu.SemaphoreType.DMA((NB,)),
            pltpu.SemaphoreType.DMA((NB,)),
            pltpu.SemaphoreType.DMA((NB,)),
        ],
    )
    def k(idx_hbm, tab_hbm, out_hbm, idx_v, rows_v, s_v, t_v,
          isem, gsem, osem):
        wid = lax.axis_index("s") * NC + lax.axis_index("c")
        blk0 = wid * per_w

        def out_hslice(c):
            blk = blk0 + c
            h = blk // n_bt
            bt = lax.rem(blk, n_bt)
            return out_hbm.at[h, pl.ds(0, D // 8), pl.ds(bt, 1),
                              pl.ds(0, 8), pl.ds(0, BLK)]

        def start_idx(c, b):
            pltpu.async_copy(idx_hbm.at[blk0 + c], idx_v.at[b], isem.at[b])

        def wait_idx(c, b):
            pltpu.make_async_copy(idx_hbm.at[blk0 + c], idx_v.at[b],
                                  isem.at[b]).wait()

        def start_gather(b):
            pltpu.async_copy(tab_hbm.at[idx_v.at[b]], rows_v.at[b],
                             gsem.at[b])

        def wait_gather(b):
            pltpu.make_async_copy(tab_hbm.at[idx_v.at[b]], rows_v.at[b],
                                  gsem.at[b]).wait()

        def start_out(c, b):
            pltpu.async_copy(t_v.at[b], out_hslice(c), osem.at[b])

        def wait_out(c, b):
            pltpu.make_async_copy(t_v.at[b], out_hslice(c),
                                  osem.at[b]).wait()

        for b in range(NB):
            start_idx(b, b)
        wait_idx(0, 0)
        start_gather(0)

        lane = lax.broadcasted_iota(jnp.int32, (L,), 0)

        def blk_body(g, carry):
            b = lax.rem(g, NB)

            @pl.when(g + 1 < per_w)
            def _next_gather():
                b1 = lax.rem(g + 1, NB)
                wait_idx(g + 1, b1)
                start_gather(b1)

            wait_gather(b)

            @pl.when(g >= NB)
            def _reuse():
                wait_out(g - NB, b)

            scales = []
            rows16 = []
            for bg in range(BG):
                idx16 = idx_v[b, pl.ds(bg * L, L)]
                scales.append(jnp.where(idx16 == 0, 0.0, 1.0))
                rows16.append(bg * L + lane)
            rows_b = rows_v.at[b]

            def pass1(d, dvec):
                vs = [plsc.load_gather(rows_b, [rows16[bg], dvec])
                      for bg in range(BG)]
                for bg in range(BG):
                    s_v[d, pl.ds(bg * L, L)] = vs[bg] * scales[bg]
                return (dvec + 1) & (D - 1)

            lax.fori_loop(0, D, pass1, lane, unroll=8)

            def pass2(c, cvec):
                vs = [plsc.load_gather(s_v, [cvec, rows16[bg]])
                      for bg in range(BG)]
                for bg in range(BG):
                    t_v[b, c // 8, 0, lax.rem(c, 8), pl.ds(bg * L, L)] = (
                        vs[bg])
                return (cvec + 1) & (D - 1)

            lax.fori_loop(0, D, pass2, (D - lane) & (D - 1), unroll=8)

            start_out(g, b)

            @pl.when(g + NB < per_w)
            def _next_idx():
                start_idx(g + NB, b)

            return carry

        lax.fori_loop(0, per_w, blk_body, 0)

        for c in range(per_w - NB, per_w):
            wait_out(c, c % NB)

    return k(idx2d, table)


def kernel(event, table):
    nb, nh = event.shape
    idx2d = event.T.reshape(nb * nh // BLK, BLK)
    out5 = _sc_gather(idx2d, table)
    return out5.transpose(2, 4, 0, 1, 3).reshape(nb, nh, D)


# speedup vs baseline: 2.6478x; 1.0078x over previous
import functools

import jax
import jax.numpy as jnp
from jax import lax
from jax.experimental import pallas as pl
from jax.experimental.pallas import tpu as pltpu
from jax.experimental.pallas import tpu_sc as plsc

D = 64
L = 16
NC = 2
NS = 16
NW = NC * NS

BLK = 128
NB = 3
BG = BLK // L


@jax.jit
def _sc_gather(idx2d, table):
    n_blocks, _ = idx2d.shape
    n_h = n_blocks * BLK // 16384
    n_bt = 16384 // BLK
    per_w = n_blocks // NW
    mesh = plsc.VectorSubcoreMesh(core_axis_name="c", subcore_axis_name="s")

    @functools.partial(
        pl.kernel,
        out_type=jax.ShapeDtypeStruct((n_h, D // 8, n_bt, 8, BLK),
                                      jnp.float32),
        mesh=mesh,
        compiler_params=pltpu.CompilerParams(
            needs_layout_passes=False, use_tc_tiling_on_sc=False),
        scratch_types=[
            pltpu.VMEM((NB, BLK), jnp.int32),
            pltpu.VMEM((NB, BLK, D), jnp.float32),
            pltpu.VMEM((D, BLK), jnp.float32),
            pltpu.VMEM((NB, D // 8, 1, 8, BLK), jnp.float32),
            pltpu.SemaphoreType.DMA((NB,)),
            pltpu.SemaphoreType.DMA((NB,)),
            pltpu.SemaphoreType.DMA((NB,)),
        ],
    )
    def k(idx_hbm, tab_hbm, out_hbm, idx_v, rows_v, s_v, t_v,
          isem, gsem, osem):
        wid = lax.axis_index("s") * NC + lax.axis_index("c")
        blk0 = wid * per_w

        def out_hslice(c):
            blk = blk0 + c
            h = blk // n_bt
            bt = lax.rem(blk, n_bt)
            return out_hbm.at[h, pl.ds(0, D // 8), pl.ds(bt, 1),
                              pl.ds(0, 8), pl.ds(0, BLK)]

        def start_idx(c, b):
            pltpu.async_copy(idx_hbm.at[blk0 + c], idx_v.at[b], isem.at[b])

        def wait_idx(c, b):
            pltpu.make_async_copy(idx_hbm.at[blk0 + c], idx_v.at[b],
                                  isem.at[b]).wait()

        def start_gather(b):
            pltpu.async_copy(tab_hbm.at[idx_v.at[b]], rows_v.at[b],
                             gsem.at[b])

        def wait_gather(b):
            pltpu.make_async_copy(tab_hbm.at[idx_v.at[b]], rows_v.at[b],
                                  gsem.at[b]).wait()

        def start_out(c, b):
            pltpu.async_copy(t_v.at[b], out_hslice(c), osem.at[b])

        def wait_out(c, b):
            pltpu.make_async_copy(t_v.at[b], out_hslice(c),
                                  osem.at[b]).wait()

        for b in range(NB):
            start_idx(b, b)
        wait_idx(0, 0)
        start_gather(0)

        lane = lax.broadcasted_iota(jnp.int32, (L,), 0)

        def blk_body(g, carry):
            b = lax.rem(g, NB)

            @pl.when(g + 1 < per_w)
            def _next_gather():
                b1 = lax.rem(g + 1, NB)
                wait_idx(g + 1, b1)
                start_gather(b1)

            wait_gather(b)

            @pl.when(g >= NB)
            def _reuse():
                wait_out(g - NB, b)

            scales = []
            rows16 = []
            for bg in range(BG):
                idx16 = idx_v[b, pl.ds(bg * L, L)]
                scales.append(jnp.where(idx16 == 0, 0.0, 1.0))
                rows16.append(bg * L + lane)
            rows_b = rows_v.at[b]

            def p1(d, dvec):
                vs = [plsc.load_gather(rows_b, [rows16[bg], dvec])
                      for bg in range(BG)]
                for bg in range(BG):
                    s_v[d, pl.ds(bg * L, L)] = vs[bg] * scales[bg]

            def p2(c, cvec):
                vs = [plsc.load_gather(s_v, [cvec, rows16[bg]])
                      for bg in range(BG)]
                for bg in range(BG):
                    t_v[b, c // 8, 0, lax.rem(c, 8), pl.ds(bg * L, L)] = (
                        vs[bg])

            def head(d, dvec):
                p1(d, dvec)
                return (dvec + 1) & (D - 1)

            dvec = lax.fori_loop(0, L, head, lane, unroll=8)

            def mid(i, carry):
                dvec, cvec = carry
                p1(L + i, dvec)
                p2(L - 1 + i, cvec)
                return ((dvec + 1) & (D - 1), (cvec + 1) & (D - 1))

            _, cvec = lax.fori_loop(
                0, D - L, mid,
                (dvec, (L - 1 - lane) & (D - 1)), unroll=8)

            def tail(i, cvec):
                c = lax.rem(D - 1 + i, D)
                p2(c, cvec)
                return (cvec + 1) & (D - 1)

            lax.fori_loop(0, L, tail, cvec, unroll=8)

            start_out(g, b)

            @pl.when(g + NB < per_w)
            def _next_idx():
                start_idx(g + NB, b)

            return carry

        lax.fori_loop(0, per_w, blk_body, 0)

        for c in range(per_w - NB, per_w):
            wait_out(c, c % NB)

    return k(idx2d, table)


def kernel(event, table):
    nb, nh = event.shape
    idx2d = event.T.reshape(nb * nh // BLK, BLK)
    out5 = _sc_gather(idx2d, table)
    return out5.transpose(2, 4, 0, 1, 3).reshape(nb, nh, D)
